# baseline (speedup 1.0000x reference)
import functools

import jax
import jax.numpy as jnp
from jax import lax
from jax.experimental import pallas as pl
from jax.experimental.pallas import tpu as pltpu
from jax.experimental.pallas import tpu_sc as plsc


def kernel(x1, x2):
    N, C, H, W = x1.shape
    rows = N * C
    a = x1.reshape(rows, H, W)
    b = x2.reshape(rows, H, W)

    info = plsc.get_sparse_core_info()
    NC, NS = info.num_cores, info.num_subcores
    NW = NC * NS
    rows_per_w = rows // NW

    mesh = plsc.VectorSubcoreMesh(core_axis_name="c", subcore_axis_name="s")

    @functools.partial(
        pl.kernel,
        mesh=mesh,
        out_type=(
            jax.ShapeDtypeStruct((rows, H, W), x1.dtype),
            jax.ShapeDtypeStruct((rows, H, W), x2.dtype),
        ),
        scratch_types=[
            pltpu.VMEM((H, W), jnp.float32),
        ],
    )
    def sc_copy(a_hbm, b_hbm, o1_hbm, o2_hbm, buf):
        wid = lax.axis_index("s") * NC + lax.axis_index("c")
        base = wid * rows_per_w

        def body(j, carry):
            r = base + j
            pltpu.sync_copy(a_hbm.at[r], buf)
            pltpu.sync_copy(buf, o1_hbm.at[r])
            pltpu.sync_copy(b_hbm.at[r], buf)
            pltpu.sync_copy(buf, o2_hbm.at[r])
            return carry

        lax.fori_loop(0, rows_per_w, body, 0)

    out1, out2 = sc_copy(a, b)
    return (out1.reshape(N, C, H, W), out2.reshape(N, C, H, W))


# speedup vs baseline: 1.1414x; 1.1414x over previous
import functools

import jax
import jax.numpy as jnp
from jax import lax
from jax.experimental import pallas as pl
from jax.experimental.pallas import tpu as pltpu
from jax.experimental.pallas import tpu_sc as plsc

_ROWS_PER_BLOCK = 32


def _tc_body(x_ref, o_ref):
    o_ref[...] = x_ref[...]


def kernel(x1, x2):
    N, C, H, W = x1.shape
    rows = N * C
    a = x1.reshape(rows, H, W)
    b = x2.reshape(rows, H, W)

    spec = pl.BlockSpec((_ROWS_PER_BLOCK, H, W), lambda i: (i, 0, 0))
    out1 = pl.pallas_call(
        _tc_body,
        grid=(rows // _ROWS_PER_BLOCK,),
        out_shape=jax.ShapeDtypeStruct((rows, H, W), x1.dtype),
        in_specs=[spec],
        out_specs=spec,
    )(a)

    info = plsc.get_sparse_core_info()
    NC, NS = info.num_cores, info.num_subcores
    NW = NC * NS
    rows_per_w = rows // NW

    mesh = plsc.VectorSubcoreMesh(core_axis_name="c", subcore_axis_name="s")

    @functools.partial(
        pl.kernel,
        mesh=mesh,
        out_type=jax.ShapeDtypeStruct((rows, H, W), x2.dtype),
        scratch_types=[
            pltpu.VMEM((H, W), jnp.float32),
        ],
    )
    def sc_copy(b_hbm, o_hbm, buf):
        wid = lax.axis_index("s") * NC + lax.axis_index("c")
        base = wid * rows_per_w

        def body(j, carry):
            r = base + j
            pltpu.sync_copy(b_hbm.at[r], buf)
            pltpu.sync_copy(buf, o_hbm.at[r])
            return carry

        lax.fori_loop(0, rows_per_w, body, 0)

    out2 = sc_copy(b)

    return (out1.reshape(N, C, H, W), out2.reshape(N, C, H, W))


# speedup vs baseline: 1.3228x; 1.1589x over previous
import jax
import jax.numpy as jnp
from jax.experimental import pallas as pl
from jax.experimental.pallas import tpu as pltpu

_ROWS_PER_BLOCK = 32


def _copy_body(x1_ref, x2_ref, o1_ref, o2_ref):
    o1_ref[...] = x1_ref[...]
    o2_ref[...] = x2_ref[...]


def kernel(x1, x2):
    N, C, H, W = x1.shape
    rows = N * C
    a = x1.reshape(rows, H, W)
    b = x2.reshape(rows, H, W)
    grid = (rows // _ROWS_PER_BLOCK,)
    spec = pl.BlockSpec((_ROWS_PER_BLOCK, H, W), lambda i: (i, 0, 0))
    out1, out2 = pl.pallas_call(
        _copy_body,
        grid=grid,
        out_shape=(
            jax.ShapeDtypeStruct((rows, H, W), x1.dtype),
            jax.ShapeDtypeStruct((rows, H, W), x2.dtype),
        ),
        in_specs=[spec, spec],
        out_specs=(spec, spec),
        compiler_params=pltpu.CompilerParams(
            dimension_semantics=("parallel",),
            vmem_limit_bytes=128 * 1024 * 1024,
        ),
    )(a, b)
    return (out1.reshape(N, C, H, W), out2.reshape(N, C, H, W))
